# baseline (speedup 1.0000x reference)
import functools

import jax
import jax.numpy as jnp
from jax.experimental import pallas as pl
from jax.experimental.pallas import tpu as pltpu

_N_TOK = 8192
_E = 16
_D = 2048
_BLK = 1024


def _router_body(x_ref, bsel_ref, W_ref, b_ref, probs_ref, kl_ref):
    i = pl.program_id(0)
    logits = jax.lax.dot_general(
        x_ref[...], W_ref[...],
        dimension_numbers=(((1,), (1,)), ((), ())),
        preferred_element_type=jnp.float32,
    ) + b_ref[...]
    m = jnp.max(logits, axis=-1, keepdims=True)
    e = jnp.exp(logits - m)
    s = jnp.sum(e, axis=-1, keepdims=True)
    probs_ref[...] = e / s
    lse = m + jnp.log(s)
    half0 = jnp.sum(logits[:, : _E // 2], axis=-1, keepdims=True)
    half1 = jnp.sum(logits[:, _E // 2 :], axis=-1, keepdims=True)
    bsel = bsel_ref[...]
    halfsum = half0 + bsel * (half1 - half0)
    part = jnp.sum(lse - 0.125 * halfsum)

    @pl.when(i == 0)
    def _init():
        kl_ref[0, 0] = 0.0

    kl_ref[0, 0] += part


@jax.jit
def kernel(x, bits, W, b):
    n = x.shape[0]
    bsel = bits.astype(jnp.float32).reshape(n, 1)
    b2 = b.astype(jnp.float32).reshape(1, _E)
    grid = (n // _BLK,)
    probs, kl = pl.pallas_call(
        _router_body,
        grid=grid,
        in_specs=[
            pl.BlockSpec((_BLK, _D), lambda i: (i, 0)),
            pl.BlockSpec((_BLK, 1), lambda i: (i, 0)),
            pl.BlockSpec((_E, _D), lambda i: (0, 0)),
            pl.BlockSpec((1, _E), lambda i: (0, 0)),
        ],
        out_specs=[
            pl.BlockSpec((_BLK, _E), lambda i: (i, 0)),
            pl.BlockSpec(memory_space=pltpu.SMEM),
        ],
        out_shape=[
            jax.ShapeDtypeStruct((n, _E), jnp.float32),
            jax.ShapeDtypeStruct((1, 1), jnp.float32),
        ],
    )(x, bsel, W, b2)
    kl_scalar = kl[0, 0] / n - jnp.log(jnp.float32(8.0))
    return (probs, kl_scalar)
